# Initial kernel scaffold; baseline (speedup 1.0000x reference)
import jax
import jax.numpy as jnp
from jax.experimental import pallas as pl


def kernel(loc_data, conf_data, priors, targets):
    raise NotImplementedError("write your pallas kernel here")


# speedup vs baseline: 4.7817x; 4.7817x over previous
import jax
import jax.numpy as jnp
from jax import lax
from jax.experimental import pallas as pl

_B = 32
_P = 8732
_C = 21
_O = 12
_R = 69
_L = 128
_PP = _R * _L
_THRESH = 0.5
_NEGPOS = 3.0
_V0, _V1 = 0.1, 0.2
_BISECT_ITERS = 50


def _body(tgt_ref, pri_ref, loc_ref, conf_ref, l_ref, c_ref, n_ref):
    b = pl.program_id(0)

    pcx = pri_ref[0]
    pcy = pri_ref[1]
    pw = pri_ref[2]
    ph = pri_ref[3]
    px1 = pcx - pw * 0.5
    py1 = pcy - ph * 0.5
    px2 = pcx + pw * 0.5
    py2 = pcy + ph * 0.5
    parea = (px2 - px1) * (py2 - py1)

    ridx = lax.broadcasted_iota(jnp.int32, (_R, _L), 0)
    cidx = lax.broadcasted_iota(jnp.int32, (_R, _L), 1)
    pidx = ridx * _L + cidx
    valid = pidx < _P

    best_ov = jnp.full((_R, _L), -1.0, dtype=jnp.float32)
    best_idx = jnp.zeros((_R, _L), dtype=jnp.int32)
    bpi = []
    for o in range(_O):
        tx1 = tgt_ref[0, o, 0]
        ty1 = tgt_ref[0, o, 1]
        tx2 = tgt_ref[0, o, 2]
        ty2 = tgt_ref[0, o, 3]
        ix = jnp.maximum(jnp.minimum(tx2, px2) - jnp.maximum(tx1, px1), 0.0)
        iy = jnp.maximum(jnp.minimum(ty2, py2) - jnp.maximum(ty1, py1), 0.0)
        inter = ix * iy
        ta = (tx2 - tx1) * (ty2 - ty1)
        iou = inter / (ta + parea - inter)
        iou = jnp.where(valid, iou, -1.0)
        mx = jnp.max(iou)
        bpi.append(jnp.min(jnp.where(iou == mx, pidx, jnp.int32(2**30))))
        upd = iou > best_ov
        best_ov = jnp.where(upd, iou, best_ov)
        best_idx = jnp.where(upd, o, best_idx)
    for o in range(_O):
        m = pidx == bpi[o]
        best_ov = jnp.where(m, 2.0, best_ov)
        best_idx = jnp.where(m, o, best_idx)

    mx1 = jnp.zeros((_R, _L), jnp.float32)
    my1 = jnp.zeros((_R, _L), jnp.float32)
    mx2 = jnp.zeros((_R, _L), jnp.float32)
    my2 = jnp.zeros((_R, _L), jnp.float32)
    lab = jnp.zeros((_R, _L), jnp.float32)
    for o in range(_O):
        m = best_idx == o
        mx1 = jnp.where(m, tgt_ref[0, o, 0], mx1)
        my1 = jnp.where(m, tgt_ref[0, o, 1], my1)
        mx2 = jnp.where(m, tgt_ref[0, o, 2], mx2)
        my2 = jnp.where(m, tgt_ref[0, o, 3], my2)
        lab = jnp.where(m, tgt_ref[0, o, 4], lab)

    conf_t = jnp.where(best_ov < _THRESH, 0.0, lab + 1.0)
    conf_t = jnp.where(valid, conf_t, 0.0)
    pos = conf_t > 0.5
    npos = jnp.sum(jnp.where(pos, 1.0, 0.0))

    g_cx = ((mx1 + mx2) * 0.5 - pcx) / (_V0 * pw)
    g_cy = ((my1 + my2) * 0.5 - pcy) / (_V0 * ph)
    g_w = jnp.log((mx2 - mx1) / pw) / _V1
    g_h = jnp.log((my2 - my1) / ph) / _V1
    loss_l = jnp.float32(0.0)
    for c, g in enumerate((g_cx, g_cy, g_w, g_h)):
        d = loc_ref[0, c] - g
        ad = jnp.abs(d)
        sl1 = jnp.where(ad < 1.0, 0.5 * d * d, ad - 0.5)
        loss_l = loss_l + jnp.sum(jnp.where(pos, sl1, 0.0))

    rmax = conf_ref[0, 0]
    for c in range(1, _C):
        rmax = jnp.maximum(rmax, conf_ref[0, c])
    ssum = jnp.zeros((_R, _L), jnp.float32)
    tgtl = jnp.zeros((_R, _L), jnp.float32)
    for c in range(_C):
        x = conf_ref[0, c]
        ssum = ssum + jnp.exp(x - rmax)
        tgtl = jnp.where(conf_t == float(c), x, tgtl)
    ce = jnp.log(ssum) + rmax - tgtl

    cem = jnp.where(pos, 0.0, ce)
    cem = jnp.where(valid, cem, -1.0)
    k = jnp.minimum(_NEGPOS * npos, jnp.float32(_P - 1))

    hi0 = jnp.max(cem)
    lo0 = jnp.float32(-0.5)

    def _bis(_, carry):
        lo, hi = carry
        mid = 0.5 * (lo + hi)
        cnt = jnp.sum(jnp.where(cem > mid, 1.0, 0.0))
        geq = cnt >= k
        return (jnp.where(geq, mid, lo), jnp.where(geq, hi, mid))

    lo, hi = lax.fori_loop(0, _BISECT_ITERS, _bis, (lo0, hi0))
    cgt = jnp.sum(jnp.where(cem > hi, 1.0, 0.0))
    need = k - cgt
    vtie = jnp.max(jnp.where((cem > lo) & (cem <= hi), cem, -1.0))
    pos_sum = jnp.sum(jnp.where(pos, ce, 0.0))
    neg_sum = jnp.sum(jnp.where(jnp.logical_and(~pos, cem > hi), ce, 0.0))
    loss_c = pos_sum + neg_sum + need * vtie

    first = b == 0
    l_ref[...] = jnp.where(first, loss_l, l_ref[0, 0] + loss_l).reshape(1, 1)
    c_ref[...] = jnp.where(first, loss_c, c_ref[0, 0] + loss_c).reshape(1, 1)
    n_ref[...] = jnp.where(first, npos, n_ref[0, 0] + npos).reshape(1, 1)

    @pl.when(b == _B - 1)
    def _():
        n = n_ref[0, 0]
        l_ref[...] = (l_ref[0, 0] / n).reshape(1, 1)
        c_ref[...] = (c_ref[0, 0] / n).reshape(1, 1)


def kernel(loc_data, conf_data, priors, targets):
    B, P, C = conf_data.shape
    pad = _PP - P
    pri_r = jnp.pad(priors, ((0, pad), (0, 0)), constant_values=0.5)
    pri_r = pri_r.T.reshape(4, _R, _L)
    loc_r = jnp.pad(loc_data, ((0, 0), (0, pad), (0, 0)))
    loc_r = loc_r.transpose(0, 2, 1).reshape(B, 4, _R, _L)
    conf_r = jnp.pad(conf_data, ((0, 0), (0, pad), (0, 0)))
    conf_r = conf_r.transpose(0, 2, 1).reshape(B, C, _R, _L)

    out_shapes = [jax.ShapeDtypeStruct((1, 1), jnp.float32)] * 3
    scalar_spec = pl.BlockSpec((1, 1), lambda b: (0, 0))
    loss_l, loss_c, _ = pl.pallas_call(
        _body,
        grid=(B,),
        in_specs=[
            pl.BlockSpec((1, _O, 5), lambda b: (b, 0, 0)),
            pl.BlockSpec((4, _R, _L), lambda b: (0, 0, 0)),
            pl.BlockSpec((1, 4, _R, _L), lambda b: (b, 0, 0, 0)),
            pl.BlockSpec((1, C, _R, _L), lambda b: (b, 0, 0, 0)),
        ],
        out_specs=[scalar_spec, scalar_spec, scalar_spec],
        out_shape=out_shapes,
    )(targets, pri_r, loc_r, conf_r)
    return loss_l[0, 0], loss_c[0, 0]
